# baseline (speedup 1.0000x reference)
import functools

import jax
import jax.numpy as jnp
from jax import lax
from jax.experimental import pallas as pl
from jax.experimental.pallas import tpu as pltpu
from jax.experimental.pallas import tpu_sc as plsc

_NC = 2
_NS = 16
_NW = _NC * _NS

_NEG_INF = float("-inf")


def _sc_mesh():
    return plsc.VectorSubcoreMesh(core_axis_name="c", subcore_axis_name="s")


def _prep1_body(h_ref, wd_ref, ws_ref, pd_ref, ps_ref):
    hb = h_ref[...]
    pd_ref[...] = jnp.dot(hb, wd_ref[...], preferred_element_type=jnp.float32)
    ps_ref[...] = jnp.dot(hb, ws_ref[...], preferred_element_type=jnp.float32)


def _prep1(h, wd, ws):
    n = h.shape[0]
    blk = 2000
    assert n % blk == 0
    return pl.pallas_call(
        _prep1_body,
        grid=(n // blk,),
        in_specs=[
            pl.BlockSpec((blk, 128), lambda i: (i, 0)),
            pl.BlockSpec((128, 128), lambda i: (0, 0)),
            pl.BlockSpec((128, 128), lambda i: (0, 0)),
        ],
        out_specs=[
            pl.BlockSpec((blk, 128), lambda i: (i, 0)),
            pl.BlockSpec((blk, 128), lambda i: (i, 0)),
        ],
        out_shape=[
            jax.ShapeDtypeStruct((n, 128), jnp.float32),
            jax.ShapeDtypeStruct((n, 128), jnp.float32),
        ],
    )(h, wd, ws)


def _prep2_body(p0_ref, p1_ref, wd_ref, ws_ref, pd_ref, ps_ref):
    hm = jnp.maximum(p0_ref[...], p1_ref[...])
    hm = jnp.where(hm == _NEG_INF, 0.0, hm)
    pd_ref[...] = jnp.dot(hm, wd_ref[...], preferred_element_type=jnp.float32)
    ps_ref[...] = jnp.dot(hm, ws_ref[...], preferred_element_type=jnp.float32)


def _prep2(p0, p1, wd, ws):
    n = p0.shape[0]
    blk = 2000
    assert n % blk == 0
    return pl.pallas_call(
        _prep2_body,
        grid=(n // blk,),
        in_specs=[
            pl.BlockSpec((blk, 128), lambda i: (i, 0)),
            pl.BlockSpec((blk, 128), lambda i: (i, 0)),
            pl.BlockSpec((128, 128), lambda i: (0, 0)),
            pl.BlockSpec((128, 128), lambda i: (0, 0)),
        ],
        out_specs=[
            pl.BlockSpec((blk, 128), lambda i: (i, 0)),
            pl.BlockSpec((blk, 128), lambda i: (i, 0)),
        ],
        out_shape=[
            jax.ShapeDtypeStruct((n, 128), jnp.float32),
            jax.ShapeDtypeStruct((n, 128), jnp.float32),
        ],
    )(p0, p1, wd, ws)


def _mlp_body(u_ref, ef_ref, wef_ref, b1_ref, w2_ref, b2_ref, out_ref):
    ef = ef_ref[...]
    wef = wef_ref[...]
    x = u_ref[...] + b1_ref[...]
    x = x + ef[:, 0:1] * wef[0:1, :]
    x = x + ef[:, 1:2] * wef[1:2, :]
    x = x + ef[:, 2:3] * wef[2:3, :]
    x = jnp.maximum(x, 0.0)
    out_ref[...] = (
        jnp.dot(x, w2_ref[...], preferred_element_type=jnp.float32) + b2_ref[...]
    )


def _mlp(u, ef, wef, b1, w2, b2):
    e = u.shape[0]
    blk = 2000
    assert e % blk == 0
    return pl.pallas_call(
        _mlp_body,
        grid=(e // blk,),
        in_specs=[
            pl.BlockSpec((blk, 128), lambda i: (i, 0)),
            pl.BlockSpec((blk, 3), lambda i: (i, 0)),
            pl.BlockSpec((3, 128), lambda i: (0, 0)),
            pl.BlockSpec((1, 128), lambda i: (0, 0)),
            pl.BlockSpec((128, 128), lambda i: (0, 0)),
            pl.BlockSpec((1, 128), lambda i: (0, 0)),
        ],
        out_specs=pl.BlockSpec((blk, 128), lambda i: (i, 0)),
        out_shape=jax.ShapeDtypeStruct((e, 128), jnp.float32),
    )(u, ef, wef, b1.reshape(1, 128), w2, b2.reshape(1, 128))


def _final_body(p0_ref, p1_ref, wr_ref, br_ref, o_ref):
    hm = jnp.maximum(p0_ref[...], p1_ref[...])
    hm = jnp.where(hm == _NEG_INF, 0.0, hm)
    o_ref[...] = (
        jnp.dot(hm, wr_ref[...], preferred_element_type=jnp.float32) + br_ref[...]
    )


def _final(p0, p1, wr, br):
    n = p0.shape[0]
    return pl.pallas_call(
        _final_body,
        out_shape=jax.ShapeDtypeStruct((n, 3), jnp.float32),
    )(p0, p1, wr, br.reshape(1, 3))


def _sc_gather_add(pd, ps, dst, src):
    n_edges = dst.shape[0]
    assert n_edges % _NW == 0
    per_w = n_edges // _NW
    c_sz = 128
    full = per_w // c_sz
    tail = per_w % c_sz
    assert tail % 8 == 0

    @functools.partial(
        pl.kernel,
        out_type=jax.ShapeDtypeStruct((n_edges, 128), jnp.float32),
        mesh=_sc_mesh(),
        compiler_params=pltpu.CompilerParams(needs_layout_passes=False),
        scratch_types=[
            pltpu.VMEM((c_sz,), jnp.int32),
            pltpu.VMEM((c_sz,), jnp.int32),
            pltpu.VMEM((c_sz, 128), jnp.float32),
            pltpu.SemaphoreType.DMA,
        ],
    )
    def k(pd_hbm, ps_hbm, dst_hbm, src_hbm, u_hbm, didx, sidx, rows, sem):
        wid = lax.axis_index("s") * _NC + lax.axis_index("c")
        base = wid * per_w

        def chunk(off, size, dbuf, sbuf, rbuf):
            pltpu.sync_copy(dst_hbm.at[pl.ds(off, size)], dbuf)
            pltpu.sync_copy(src_hbm.at[pl.ds(off, size)], sbuf)
            pltpu.async_copy(pd_hbm.at[dbuf], rbuf, sem).wait()
            pltpu.async_copy(ps_hbm.at[sbuf], rbuf, sem, add=True).wait()
            pltpu.sync_copy(rbuf, u_hbm.at[pl.ds(off, size)])

        def body(i, _):
            chunk(base + i * c_sz, c_sz, didx, sidx, rows)
            return 0

        lax.fori_loop(0, full, body, 0)
        if tail:
            chunk(
                base + full * c_sz,
                tail,
                didx.at[pl.ds(0, tail)],
                sidx.at[pl.ds(0, tail)],
                rows.at[pl.ds(0, tail)],
            )

    return k(pd, ps, dst, src)


def _sc_scatter_max(m, dst, n_pad):
    n_edges = dst.shape[0]
    assert n_edges % _NC == 0
    half = n_edges // _NC
    rows_per_tile = n_pad // _NS
    ch = 2048
    full = half // ch
    tail = half % ch
    assert tail % 16 == 0
    gc = 128

    @functools.partial(
        pl.kernel,
        out_type=jax.ShapeDtypeStruct((_NC * n_pad * 128,), jnp.float32),
        mesh=_sc_mesh(),
        compiler_params=pltpu.CompilerParams(needs_layout_passes=False),
        scratch_types=[
            pltpu.VMEM((ch,), jnp.int32),
            pltpu.VMEM((ch + 16,), jnp.int32),
            pltpu.VMEM((ch + 16,), jnp.int32),
            pltpu.VMEM((gc,), jnp.int32),
            pltpu.VMEM((gc,), jnp.int32),
            pltpu.HBM((_NW, gc), jnp.int32),
            pltpu.VMEM((gc, 128), jnp.float32),
            pltpu.VMEM((rows_per_tile * 128,), jnp.float32),
            pltpu.SemaphoreType.DMA,
        ],
    )
    def k(m_hbm, dst_hbm, out_hbm, dbuf, idsbuf, dstbuf, gidx, gidx2, hstage, rows, acc, sem):
        c = lax.axis_index("c")
        s = lax.axis_index("s")
        wid = s * _NC + c
        lo = s * rows_per_tile
        hi = lo + rows_per_tile
        ebase = c * half
        iota16 = lax.iota(jnp.int32, 16)

        neg = jnp.full((16,), _NEG_INF, jnp.float32)

        def initacc(i, _):
            acc[pl.ds(i * 16, 16)] = neg
            return 0

        lax.fori_loop(0, rows_per_tile * 128 // 16, initacc, 0)

        zero16 = jnp.zeros((16,), jnp.int32)

        def initids(i, _):
            idsbuf[pl.ds(i * 16, 16)] = zero16
            return 0

        lax.fori_loop(0, (ch + 16) // 16, initids, 0)

        lov = jnp.full((16,), lo, jnp.int32)
        hiv = jnp.full((16,), hi, jnp.int32)

        def scan_chunk(chbase, size):
            pltpu.sync_copy(
                dst_hbm.at[pl.ds(ebase + chbase, size)], dbuf.at[pl.ds(0, size)]
            )

            def svec(v, cnt):
                d = dbuf[pl.ds(v * 16, 16)]
                eid = jnp.full((16,), ebase + chbase + v * 16, jnp.int32) + iota16
                mask = (d >= lov) & (d < hiv)
                cs = plsc.cumsum(mask.astype(jnp.int32))
                pos = jnp.full((16,), cnt - 1, jnp.int32) + cs
                plsc.store_scatter(idsbuf, [pos], eid, mask=mask)
                plsc.store_scatter(dstbuf, [pos], d, mask=mask)
                return cnt + cs[15]

            return lax.fori_loop(0, size // 16, svec, jnp.int32(0))

        def process(cnt):
            nsub = (cnt + gc - 1) // gc

            def sub(k2, _):
                sbase = k2 * gc

                def cpi(j, _):
                    gidx[pl.ds(j * 16, 16)] = idsbuf[pl.ds(sbase + j * 16, 16)]
                    return 0

                lax.fori_loop(0, gc // 16, cpi, 0)
                pltpu.sync_copy(gidx, hstage.at[wid])
                pltpu.sync_copy(hstage.at[wid], gidx2)
                pltpu.async_copy(m_hbm.at[gidx2], rows, sem).wait()
                napply = jnp.minimum(cnt - sbase, gc)

                def apply(r, _):
                    dv = dstbuf[pl.ds(sbase + r, 16)][0]
                    ab = (dv - lo) * 128
                    for j2 in range(8):
                        a = acc[pl.ds(ab + j2 * 16, 16)]
                        b = rows[r, pl.ds(j2 * 16, 16)]
                        acc[pl.ds(ab + j2 * 16, 16)] = jnp.maximum(a, b)
                    return 0

                lax.fori_loop(0, napply, apply, 0)
                return 0

            lax.fori_loop(0, nsub, sub, 0)

        def chunk_body(i, _):
            cnt = scan_chunk(i * ch, ch)
            _BISECT = False
            if not _BISECT:
                @pl.when(cnt > 0)
                def _():
                    process(cnt)

            return 0

        lax.fori_loop(0, full, chunk_body, 0)
        if tail:
            cnt = scan_chunk(full * ch, tail)
            _BISECT = False
            if not _BISECT:
                @pl.when(cnt > 0)
                def _():
                    process(cnt)

        obase = (c * n_pad + lo) * 128
        pltpu.sync_copy(acc, out_hbm.at[pl.ds(obase, rows_per_tile * 128)])

    return k(m, dst)


def kernel(h, edge_index, edge_features, W1a, b1a, W2a, b2a, W1b, b1b, W2b, b2b, Wr, br):
    n = h.shape[0]
    n_pad = (n + _NS - 1) // _NS * _NS

    src1 = edge_index[0]
    dst1 = edge_index[1]
    src2 = edge_index[2]
    dst2 = edge_index[3]
    ef0 = edge_features[0::2]
    ef1 = edge_features[1::2]

    pd1, ps1 = _prep1(h, W1a[:128], W1a[128:256])
    u1 = _sc_gather_add(pd1, ps1, dst1, src1)
    m1 = _mlp(u1, ef0, W1a[256:], b1a, W2a, b2a)
    part1 = _sc_scatter_max(m1, dst1, n_pad).reshape(_NC, n_pad, 128)

    pd2, ps2 = _prep2(part1[0], part1[1], W1b[:128], W1b[128:256])
    u2 = _sc_gather_add(pd2, ps2, dst2, src2)
    m2 = _mlp(u2, ef1, W1b[256:], b1b, W2b, b2b)
    part2 = _sc_scatter_max(m2, dst2, n_pad).reshape(_NC, n_pad, 128)

    sel0 = part2[0, 8 : n - 1 : 9]
    sel1 = part2[1, 8 : n - 1 : 9]
    rows = sel0.shape[0]
    rows_pad = (rows + 7) // 8 * 8
    pad = rows_pad - rows
    sel0 = jnp.pad(sel0, ((0, pad), (0, 0)))
    sel1 = jnp.pad(sel1, ((0, pad), (0, 0)))
    o = _final(sel0, sel1, Wr, br)
    return o[:rows]


# speedup vs baseline: 4.1744x; 4.1744x over previous
import functools

import jax
import jax.numpy as jnp
from jax import lax
from jax.experimental import pallas as pl
from jax.experimental.pallas import tpu as pltpu
from jax.experimental.pallas import tpu_sc as plsc

_NC = 2
_NS = 16
_NW = _NC * _NS

_NEG_INF = float("-inf")


def _sc_mesh():
    return plsc.VectorSubcoreMesh(core_axis_name="c", subcore_axis_name="s")


def _prep1_body(h_ref, wd_ref, ws_ref, pd_ref, ps_ref):
    hb = h_ref[...]
    pd_ref[...] = jnp.dot(hb, wd_ref[...], preferred_element_type=jnp.float32)
    ps_ref[...] = jnp.dot(hb, ws_ref[...], preferred_element_type=jnp.float32)


def _prep1(h, wd, ws):
    n = h.shape[0]
    blk = 2000
    assert n % blk == 0
    return pl.pallas_call(
        _prep1_body,
        grid=(n // blk,),
        in_specs=[
            pl.BlockSpec((blk, 128), lambda i: (i, 0)),
            pl.BlockSpec((128, 128), lambda i: (0, 0)),
            pl.BlockSpec((128, 128), lambda i: (0, 0)),
        ],
        out_specs=[
            pl.BlockSpec((blk, 128), lambda i: (i, 0)),
            pl.BlockSpec((blk, 128), lambda i: (i, 0)),
        ],
        out_shape=[
            jax.ShapeDtypeStruct((n, 128), jnp.float32),
            jax.ShapeDtypeStruct((n, 128), jnp.float32),
        ],
    )(h, wd, ws)


def _prep2_body(p0_ref, p1_ref, wd_ref, ws_ref, pd_ref, ps_ref):
    hm = jnp.maximum(p0_ref[...], p1_ref[...])
    hm = jnp.where(hm == _NEG_INF, 0.0, hm)
    pd_ref[...] = jnp.dot(hm, wd_ref[...], preferred_element_type=jnp.float32)
    ps_ref[...] = jnp.dot(hm, ws_ref[...], preferred_element_type=jnp.float32)


def _prep2(p0, p1, wd, ws):
    n = p0.shape[0]
    blk = 2000
    assert n % blk == 0
    return pl.pallas_call(
        _prep2_body,
        grid=(n // blk,),
        in_specs=[
            pl.BlockSpec((blk, 128), lambda i: (i, 0)),
            pl.BlockSpec((blk, 128), lambda i: (i, 0)),
            pl.BlockSpec((128, 128), lambda i: (0, 0)),
            pl.BlockSpec((128, 128), lambda i: (0, 0)),
        ],
        out_specs=[
            pl.BlockSpec((blk, 128), lambda i: (i, 0)),
            pl.BlockSpec((blk, 128), lambda i: (i, 0)),
        ],
        out_shape=[
            jax.ShapeDtypeStruct((n, 128), jnp.float32),
            jax.ShapeDtypeStruct((n, 128), jnp.float32),
        ],
    )(p0, p1, wd, ws)


def _mlp_body(u_ref, ef_ref, wef_ref, b1_ref, w2_ref, b2_ref, out_ref):
    ef = ef_ref[...]
    wef = wef_ref[...]
    x = u_ref[...] + b1_ref[...]
    x = x + ef[:, 0:1] * wef[0:1, :]
    x = x + ef[:, 1:2] * wef[1:2, :]
    x = x + ef[:, 2:3] * wef[2:3, :]
    x = jnp.maximum(x, 0.0)
    out_ref[...] = (
        jnp.dot(x, w2_ref[...], preferred_element_type=jnp.float32) + b2_ref[...]
    )


def _mlp(u, ef, wef, b1, w2, b2):
    e = u.shape[0]
    blk = 2000
    assert e % blk == 0
    return pl.pallas_call(
        _mlp_body,
        grid=(e // blk,),
        in_specs=[
            pl.BlockSpec((blk, 128), lambda i: (i, 0)),
            pl.BlockSpec((blk, 3), lambda i: (i, 0)),
            pl.BlockSpec((3, 128), lambda i: (0, 0)),
            pl.BlockSpec((1, 128), lambda i: (0, 0)),
            pl.BlockSpec((128, 128), lambda i: (0, 0)),
            pl.BlockSpec((1, 128), lambda i: (0, 0)),
        ],
        out_specs=pl.BlockSpec((blk, 128), lambda i: (i, 0)),
        out_shape=jax.ShapeDtypeStruct((e, 128), jnp.float32),
    )(u, ef, wef, b1.reshape(1, 128), w2, b2.reshape(1, 128))


def _final_body(p0_ref, p1_ref, wr_ref, br_ref, o_ref):
    hm = jnp.maximum(p0_ref[...], p1_ref[...])
    hm = jnp.where(hm == _NEG_INF, 0.0, hm)
    o_ref[...] = (
        jnp.dot(hm, wr_ref[...], preferred_element_type=jnp.float32) + br_ref[...]
    )


def _final(p0, p1, wr, br):
    n = p0.shape[0]
    return pl.pallas_call(
        _final_body,
        out_shape=jax.ShapeDtypeStruct((n, 3), jnp.float32),
    )(p0, p1, wr, br.reshape(1, 3))


def _sc_gather_add(pd, ps, dst, src):
    n_edges = dst.shape[0]
    assert n_edges % _NW == 0
    per_w = n_edges // _NW
    c_sz = 128
    full = per_w // c_sz
    tail = per_w % c_sz
    assert tail % 8 == 0

    assert full % 2 == 0

    @functools.partial(
        pl.kernel,
        out_type=jax.ShapeDtypeStruct((n_edges, 128), jnp.float32),
        mesh=_sc_mesh(),
        compiler_params=pltpu.CompilerParams(needs_layout_passes=False),
        scratch_types=[
            pltpu.VMEM((c_sz,), jnp.int32),
            pltpu.VMEM((c_sz,), jnp.int32),
            pltpu.VMEM((c_sz, 128), jnp.float32),
            pltpu.VMEM((c_sz,), jnp.int32),
            pltpu.VMEM((c_sz,), jnp.int32),
            pltpu.VMEM((c_sz, 128), jnp.float32),
            pltpu.SemaphoreType.DMA,
            pltpu.SemaphoreType.DMA,
        ],
    )
    def k(pd_hbm, ps_hbm, dst_hbm, src_hbm, u_hbm,
          didx0, sidx0, rows0, didx1, sidx1, rows1, sem0, sem1):
        wid = lax.axis_index("s") * _NC + lax.axis_index("c")
        base = wid * per_w

        def pair(i, _):
            off0 = base + (2 * i) * c_sz
            off1 = off0 + c_sz
            pltpu.sync_copy(dst_hbm.at[pl.ds(off0, c_sz)], didx0)
            pltpu.sync_copy(src_hbm.at[pl.ds(off0, c_sz)], sidx0)
            g0 = pltpu.async_copy(pd_hbm.at[didx0], rows0, sem0)
            pltpu.sync_copy(dst_hbm.at[pl.ds(off1, c_sz)], didx1)
            pltpu.sync_copy(src_hbm.at[pl.ds(off1, c_sz)], sidx1)
            g1 = pltpu.async_copy(pd_hbm.at[didx1], rows1, sem1)
            g0.wait()
            a0 = pltpu.async_copy(ps_hbm.at[sidx0], rows0, sem0, add=True)
            g1.wait()
            a1 = pltpu.async_copy(ps_hbm.at[sidx1], rows1, sem1, add=True)
            a0.wait()
            pltpu.sync_copy(rows0, u_hbm.at[pl.ds(off0, c_sz)])
            a1.wait()
            pltpu.sync_copy(rows1, u_hbm.at[pl.ds(off1, c_sz)])
            return 0

        lax.fori_loop(0, full // 2, pair, 0)
        if tail:
            off = base + full * c_sz
            dbuf = didx0.at[pl.ds(0, tail)]
            sbuf = sidx0.at[pl.ds(0, tail)]
            rbuf = rows0.at[pl.ds(0, tail)]
            pltpu.sync_copy(dst_hbm.at[pl.ds(off, tail)], dbuf)
            pltpu.sync_copy(src_hbm.at[pl.ds(off, tail)], sbuf)
            pltpu.async_copy(pd_hbm.at[dbuf], rbuf, sem0).wait()
            pltpu.async_copy(ps_hbm.at[sbuf], rbuf, sem0, add=True).wait()
            pltpu.sync_copy(rbuf, u_hbm.at[pl.ds(off, tail)])

    return k(pd, ps, dst, src)


def _sc_scatter_prep(dst, n_pad):
    n_edges = dst.shape[0]
    assert n_edges % _NC == 0
    half = n_edges // _NC
    rows_per_tile = n_pad // _NS
    ch = 2048
    full = half // ch
    tail = half % ch
    assert tail % 16 == 0
    cap = half + 2 * (ch + 16)
    assert cap % 8 == 0

    @functools.partial(
        pl.kernel,
        out_type=(
            jax.ShapeDtypeStruct((_NW * cap,), jnp.int32),
            jax.ShapeDtypeStruct((_NW * cap,), jnp.int32),
            jax.ShapeDtypeStruct((_NW * 16,), jnp.int32),
        ),
        mesh=_sc_mesh(),
        compiler_params=pltpu.CompilerParams(needs_layout_passes=False),
        scratch_types=[
            pltpu.VMEM((ch,), jnp.int32),
            pltpu.VMEM((ch + 16,), jnp.int32),
            pltpu.VMEM((ch + 16,), jnp.int32),
            pltpu.VMEM((16,), jnp.int32),
        ],
    )
    def k(dst_hbm, ids_hbm, dsts_hbm, cnts_hbm, dbuf, idsbuf, dstbuf, cbuf):
        c = lax.axis_index("c")
        s = lax.axis_index("s")
        wid = s * _NC + c
        lo = s * rows_per_tile
        hi = lo + rows_per_tile
        ebase = c * half
        iota16 = lax.iota(jnp.int32, 16)
        lov = jnp.full((16,), lo, jnp.int32)
        hiv = jnp.full((16,), hi, jnp.int32)
        zero16 = jnp.zeros((16,), jnp.int32)

        def initids(i, _):
            idsbuf[pl.ds(i * 16, 16)] = zero16
            dstbuf[pl.ds(i * 16, 16)] = lov
            return 0

        lax.fori_loop(0, (ch + 16) // 16, initids, 0)

        def scan_chunk(chbase, size):
            pltpu.sync_copy(
                dst_hbm.at[pl.ds(ebase + chbase, size)], dbuf.at[pl.ds(0, size)]
            )

            def svec(v, cnt):
                d = dbuf[pl.ds(v * 16, 16)]
                eid = jnp.full((16,), ebase + chbase + v * 16, jnp.int32) + iota16
                mask = (d >= lov) & (d < hiv)
                cs = plsc.cumsum(mask.astype(jnp.int32))
                pos = jnp.full((16,), cnt - 1, jnp.int32) + cs
                plsc.store_scatter(idsbuf, [pos], eid, mask=mask)
                plsc.store_scatter(dstbuf, [pos], d, mask=mask)
                return cnt + cs[15]

            return lax.fori_loop(0, size // 16, svec, jnp.int32(0))

        def do_chunk(chbase, size, cursor):
            cnt = scan_chunk(chbase, size)

            @pl.when(cnt > 0)
            def _():
                last_id = jnp.full((16,), idsbuf[pl.ds(cnt - 1, 16)][0], jnp.int32)
                last_d = jnp.full((16,), dstbuf[pl.ds(cnt - 1, 16)][0], jnp.int32)
                posp = jnp.full((16,), cnt, jnp.int32) + iota16
                ones = jnp.full((16,), True, jnp.bool_)
                plsc.store_scatter(idsbuf, [posp], last_id, mask=ones)
                plsc.store_scatter(dstbuf, [posp], last_d, mask=ones)
                off = pl.multiple_of(wid * cap + cursor, 8)
                pltpu.sync_copy(idsbuf, ids_hbm.at[pl.ds(off, ch + 16)])
                pltpu.sync_copy(dstbuf, dsts_hbm.at[pl.ds(off, ch + 16)])

            cnt8 = (cnt + 7) & ~7
            return cursor + cnt8

        def chunk_body(i, cursor):
            return do_chunk(i * ch, ch, cursor)

        cursor = lax.fori_loop(0, full, chunk_body, jnp.int32(0))
        if tail:
            cursor = do_chunk(full * ch, tail, cursor)

        def zeroids(i, _):
            idsbuf[pl.ds(i * 16, 16)] = zero16
            dstbuf[pl.ds(i * 16, 16)] = lov
            return 0

        lax.fori_loop(0, (ch + 16) // 16, zeroids, 0)
        off = pl.multiple_of(wid * cap + cursor, 8)
        pltpu.sync_copy(idsbuf, ids_hbm.at[pl.ds(off, ch + 16)])
        pltpu.sync_copy(dstbuf, dsts_hbm.at[pl.ds(off, ch + 16)])
        cbuf[pl.ds(0, 16)] = jnp.full((16,), cursor, jnp.int32)
        pltpu.sync_copy(cbuf, cnts_hbm.at[pl.ds(wid * 16, 16)])

    return k(dst)


def _sc_scatter_apply(m, ids, dsts, cnts, n_pad, cap):
    rows_per_tile = n_pad // _NS
    gc = 128

    @functools.partial(
        pl.kernel,
        out_type=jax.ShapeDtypeStruct((_NC * n_pad * 128,), jnp.float32),
        mesh=_sc_mesh(),
        compiler_params=pltpu.CompilerParams(needs_layout_passes=False),
        scratch_types=[
            pltpu.VMEM((gc,), jnp.int32),
            pltpu.VMEM((gc + 16,), jnp.int32),
            pltpu.VMEM((16,), jnp.int32),
            pltpu.VMEM((gc, 128), jnp.float32),
            pltpu.VMEM((rows_per_tile * 128,), jnp.float32),
            pltpu.SemaphoreType.DMA,
        ],
    )
    def k(m_hbm, ids_hbm, dsts_hbm, cnts_hbm, out_hbm, gidx, dstc, cbuf, rows, acc, sem):
        c = lax.axis_index("c")
        s = lax.axis_index("s")
        wid = s * _NC + c
        lo = s * rows_per_tile

        neg = jnp.full((16,), _NEG_INF, jnp.float32)

        def initacc(i, _):
            acc[pl.ds(i * 16, 16)] = neg
            return 0

        lax.fori_loop(0, rows_per_tile * 128 // 16, initacc, 0)

        pltpu.sync_copy(cnts_hbm.at[pl.ds(wid * 16, 16)], cbuf)
        cnt = cbuf[pl.ds(0, 16)][0]
        nsub = (cnt + gc - 1) // gc

        def sub(k2, _):
            sbase = k2 * gc
            soff = pl.multiple_of(wid * cap + sbase, 8)
            pltpu.sync_copy(ids_hbm.at[pl.ds(soff, gc)], gidx)
            pltpu.sync_copy(dsts_hbm.at[pl.ds(soff, gc)], dstc.at[pl.ds(0, gc)])
            pltpu.async_copy(m_hbm.at[gidx], rows, sem).wait()
            napply = jnp.minimum(cnt - sbase, gc)

            def apply(r, _):
                dv = dstc[pl.ds(r, 16)][0]
                ab = (dv - lo) * 128
                for j2 in range(8):
                    a = acc[pl.ds(ab + j2 * 16, 16)]
                    b = rows[r, pl.ds(j2 * 16, 16)]
                    acc[pl.ds(ab + j2 * 16, 16)] = jnp.maximum(a, b)
                return 0

            lax.fori_loop(0, napply, apply, 0)
            return 0

        lax.fori_loop(0, nsub, sub, 0)

        obase = (c * n_pad + lo) * 128
        pltpu.sync_copy(acc, out_hbm.at[pl.ds(obase, rows_per_tile * 128)])

    return k(m, ids, dsts, cnts)


def _sc_scatter_max(m, dst, n_pad):
    ids, dsts, cnts = _sc_scatter_prep(dst, n_pad)
    cap = ids.shape[0] // _NW
    return _sc_scatter_apply(m, ids, dsts, cnts, n_pad, cap)


def kernel(h, edge_index, edge_features, W1a, b1a, W2a, b2a, W1b, b1b, W2b, b2b, Wr, br):
    n = h.shape[0]
    n_pad = (n + _NS - 1) // _NS * _NS

    src1 = edge_index[0]
    dst1 = edge_index[1]
    src2 = edge_index[2]
    dst2 = edge_index[3]
    ef0 = edge_features[0::2]
    ef1 = edge_features[1::2]

    pd1, ps1 = _prep1(h, W1a[:128], W1a[128:256])
    u1 = _sc_gather_add(pd1, ps1, dst1, src1)
    m1 = _mlp(u1, ef0, W1a[256:], b1a, W2a, b2a)
    part1 = _sc_scatter_max(m1, dst1, n_pad).reshape(_NC, n_pad, 128)

    pd2, ps2 = _prep2(part1[0], part1[1], W1b[:128], W1b[128:256])
    u2 = _sc_gather_add(pd2, ps2, dst2, src2)
    m2 = _mlp(u2, ef1, W1b[256:], b1b, W2b, b2b)
    part2 = _sc_scatter_max(m2, dst2, n_pad).reshape(_NC, n_pad, 128)

    sel0 = part2[0, 8 : n - 1 : 9]
    sel1 = part2[1, 8 : n - 1 : 9]
    rows = sel0.shape[0]
    rows_pad = (rows + 7) // 8 * 8
    pad = rows_pad - rows
    sel0 = jnp.pad(sel0, ((0, pad), (0, 0)))
    sel1 = jnp.pad(sel1, ((0, pad), (0, 0)))
    o = _final(sel0, sel1, Wr, br)
    return o[:rows]


# speedup vs baseline: 4.4700x; 1.0708x over previous
import functools

import jax
import jax.numpy as jnp
from jax import lax
from jax.experimental import pallas as pl
from jax.experimental.pallas import tpu as pltpu
from jax.experimental.pallas import tpu_sc as plsc

_NC = 2
_NS = 16
_NW = _NC * _NS

_NEG_INF = float("-inf")


def _sc_mesh():
    return plsc.VectorSubcoreMesh(core_axis_name="c", subcore_axis_name="s")


def _prep1_body(h_ref, wd_ref, ws_ref, pd_ref, ps_ref):
    hb = h_ref[...]
    pd_ref[...] = jnp.dot(hb, wd_ref[...], preferred_element_type=jnp.float32)
    ps_ref[...] = jnp.dot(hb, ws_ref[...], preferred_element_type=jnp.float32)


def _prep1(h, wd, ws):
    n = h.shape[0]
    blk = 2000
    assert n % blk == 0
    return pl.pallas_call(
        _prep1_body,
        grid=(n // blk,),
        in_specs=[
            pl.BlockSpec((blk, 128), lambda i: (i, 0)),
            pl.BlockSpec((128, 128), lambda i: (0, 0)),
            pl.BlockSpec((128, 128), lambda i: (0, 0)),
        ],
        out_specs=[
            pl.BlockSpec((blk, 128), lambda i: (i, 0)),
            pl.BlockSpec((blk, 128), lambda i: (i, 0)),
        ],
        out_shape=[
            jax.ShapeDtypeStruct((n, 128), jnp.float32),
            jax.ShapeDtypeStruct((n, 128), jnp.float32),
        ],
    )(h, wd, ws)


def _prep2_body(p0_ref, p1_ref, wd_ref, ws_ref, pd_ref, ps_ref):
    hm = jnp.maximum(p0_ref[...], p1_ref[...])
    hm = jnp.where(hm == _NEG_INF, 0.0, hm)
    pd_ref[...] = jnp.dot(hm, wd_ref[...], preferred_element_type=jnp.float32)
    ps_ref[...] = jnp.dot(hm, ws_ref[...], preferred_element_type=jnp.float32)


def _prep2(p0, p1, wd, ws):
    n = p0.shape[0]
    blk = 2000
    assert n % blk == 0
    return pl.pallas_call(
        _prep2_body,
        grid=(n // blk,),
        in_specs=[
            pl.BlockSpec((blk, 128), lambda i: (i, 0)),
            pl.BlockSpec((blk, 128), lambda i: (i, 0)),
            pl.BlockSpec((128, 128), lambda i: (0, 0)),
            pl.BlockSpec((128, 128), lambda i: (0, 0)),
        ],
        out_specs=[
            pl.BlockSpec((blk, 128), lambda i: (i, 0)),
            pl.BlockSpec((blk, 128), lambda i: (i, 0)),
        ],
        out_shape=[
            jax.ShapeDtypeStruct((n, 128), jnp.float32),
            jax.ShapeDtypeStruct((n, 128), jnp.float32),
        ],
    )(p0, p1, wd, ws)


def _mlp_body(u_ref, ef_ref, wef_ref, b1_ref, w2_ref, b2_ref, out_ref):
    ef = ef_ref[...]
    wef = wef_ref[...]
    x = u_ref[...] + b1_ref[...]
    x = x + ef[:, 0:1] * wef[0:1, :]
    x = x + ef[:, 1:2] * wef[1:2, :]
    x = x + ef[:, 2:3] * wef[2:3, :]
    x = jnp.maximum(x, 0.0)
    out_ref[...] = (
        jnp.dot(x, w2_ref[...], preferred_element_type=jnp.float32) + b2_ref[...]
    )


def _mlp(u, ef, wef, b1, w2, b2):
    e = u.shape[0]
    blk = 2000
    assert e % blk == 0
    return pl.pallas_call(
        _mlp_body,
        grid=(e // blk,),
        in_specs=[
            pl.BlockSpec((blk, 128), lambda i: (i, 0)),
            pl.BlockSpec((blk, 3), lambda i: (i, 0)),
            pl.BlockSpec((3, 128), lambda i: (0, 0)),
            pl.BlockSpec((1, 128), lambda i: (0, 0)),
            pl.BlockSpec((128, 128), lambda i: (0, 0)),
            pl.BlockSpec((1, 128), lambda i: (0, 0)),
        ],
        out_specs=pl.BlockSpec((blk, 128), lambda i: (i, 0)),
        out_shape=jax.ShapeDtypeStruct((e, 128), jnp.float32),
    )(u, ef, wef, b1.reshape(1, 128), w2, b2.reshape(1, 128))


def _final_body(p0_ref, p1_ref, wr_ref, br_ref, o_ref):
    hm = jnp.maximum(p0_ref[...], p1_ref[...])
    hm = jnp.where(hm == _NEG_INF, 0.0, hm)
    o_ref[...] = (
        jnp.dot(hm, wr_ref[...], preferred_element_type=jnp.float32) + br_ref[...]
    )


def _final(p0, p1, wr, br):
    n = p0.shape[0]
    return pl.pallas_call(
        _final_body,
        out_shape=jax.ShapeDtypeStruct((n, 3), jnp.float32),
    )(p0, p1, wr, br.reshape(1, 3))


def _sc_gather_add(pd, ps, dst, src):
    n_edges = dst.shape[0]
    assert n_edges % _NW == 0
    per_w = n_edges // _NW
    c_sz = 128
    full = per_w // c_sz
    tail = per_w % c_sz
    assert tail % 8 == 0

    assert full % 2 == 0

    @functools.partial(
        pl.kernel,
        out_type=jax.ShapeDtypeStruct((n_edges, 128), jnp.float32),
        mesh=_sc_mesh(),
        compiler_params=pltpu.CompilerParams(needs_layout_passes=False),
        scratch_types=[
            pltpu.VMEM((c_sz,), jnp.int32),
            pltpu.VMEM((c_sz,), jnp.int32),
            pltpu.VMEM((c_sz, 128), jnp.float32),
            pltpu.VMEM((c_sz,), jnp.int32),
            pltpu.VMEM((c_sz,), jnp.int32),
            pltpu.VMEM((c_sz, 128), jnp.float32),
            pltpu.SemaphoreType.DMA,
            pltpu.SemaphoreType.DMA,
        ],
    )
    def k(pd_hbm, ps_hbm, dst_hbm, src_hbm, u_hbm,
          didx0, sidx0, rows0, didx1, sidx1, rows1, sem0, sem1):
        wid = lax.axis_index("s") * _NC + lax.axis_index("c")
        base = wid * per_w

        def pair(i, _):
            off0 = base + (2 * i) * c_sz
            off1 = off0 + c_sz
            pltpu.sync_copy(dst_hbm.at[pl.ds(off0, c_sz)], didx0)
            pltpu.sync_copy(src_hbm.at[pl.ds(off0, c_sz)], sidx0)
            g0 = pltpu.async_copy(pd_hbm.at[didx0], rows0, sem0)
            pltpu.sync_copy(dst_hbm.at[pl.ds(off1, c_sz)], didx1)
            pltpu.sync_copy(src_hbm.at[pl.ds(off1, c_sz)], sidx1)
            g1 = pltpu.async_copy(pd_hbm.at[didx1], rows1, sem1)
            g0.wait()
            a0 = pltpu.async_copy(ps_hbm.at[sidx0], rows0, sem0, add=True)
            g1.wait()
            a1 = pltpu.async_copy(ps_hbm.at[sidx1], rows1, sem1, add=True)
            a0.wait()
            pltpu.sync_copy(rows0, u_hbm.at[pl.ds(off0, c_sz)])
            a1.wait()
            pltpu.sync_copy(rows1, u_hbm.at[pl.ds(off1, c_sz)])
            return 0

        lax.fori_loop(0, full // 2, pair, 0)
        if tail:
            off = base + full * c_sz
            dbuf = didx0.at[pl.ds(0, tail)]
            sbuf = sidx0.at[pl.ds(0, tail)]
            rbuf = rows0.at[pl.ds(0, tail)]
            pltpu.sync_copy(dst_hbm.at[pl.ds(off, tail)], dbuf)
            pltpu.sync_copy(src_hbm.at[pl.ds(off, tail)], sbuf)
            pltpu.async_copy(pd_hbm.at[dbuf], rbuf, sem0).wait()
            pltpu.async_copy(ps_hbm.at[sbuf], rbuf, sem0, add=True).wait()
            pltpu.sync_copy(rbuf, u_hbm.at[pl.ds(off, tail)])

    return k(pd, ps, dst, src)


def _sc_scatter_prep(dst, n_pad):
    n_edges = dst.shape[0]
    assert n_edges % _NC == 0
    half = n_edges // _NC
    rows_per_tile = n_pad // _NS
    ch = 2048
    full = half // ch
    tail = half % ch
    assert tail % 16 == 0
    cap = half + 2 * (ch + 16)
    assert cap % 8 == 0

    @functools.partial(
        pl.kernel,
        out_type=(
            jax.ShapeDtypeStruct((_NW * cap,), jnp.int32),
            jax.ShapeDtypeStruct((_NW * cap,), jnp.int32),
            jax.ShapeDtypeStruct((_NW * 16,), jnp.int32),
        ),
        mesh=_sc_mesh(),
        compiler_params=pltpu.CompilerParams(needs_layout_passes=False),
        scratch_types=[
            pltpu.VMEM((ch,), jnp.int32),
            pltpu.VMEM((ch + 16,), jnp.int32),
            pltpu.VMEM((ch + 16,), jnp.int32),
            pltpu.VMEM((16,), jnp.int32),
        ],
    )
    def k(dst_hbm, ids_hbm, dsts_hbm, cnts_hbm, dbuf, idsbuf, dstbuf, cbuf):
        c = lax.axis_index("c")
        s = lax.axis_index("s")
        wid = s * _NC + c
        lo = s * rows_per_tile
        hi = lo + rows_per_tile
        ebase = c * half
        iota16 = lax.iota(jnp.int32, 16)
        lov = jnp.full((16,), lo, jnp.int32)
        hiv = jnp.full((16,), hi, jnp.int32)
        zero16 = jnp.zeros((16,), jnp.int32)

        def initids(i, _):
            idsbuf[pl.ds(i * 16, 16)] = zero16
            dstbuf[pl.ds(i * 16, 16)] = lov
            return 0

        lax.fori_loop(0, (ch + 16) // 16, initids, 0)

        def scan_chunk(chbase, size):
            pltpu.sync_copy(
                dst_hbm.at[pl.ds(ebase + chbase, size)], dbuf.at[pl.ds(0, size)]
            )

            def svec(v, cnt):
                d = dbuf[pl.ds(v * 16, 16)]
                eid = jnp.full((16,), ebase + chbase + v * 16, jnp.int32) + iota16
                mask = (d >= lov) & (d < hiv)
                cs = plsc.cumsum(mask.astype(jnp.int32))
                pos = jnp.full((16,), cnt - 1, jnp.int32) + cs
                plsc.store_scatter(idsbuf, [pos], eid, mask=mask)
                plsc.store_scatter(dstbuf, [pos], d, mask=mask)
                return cnt + cs[15]

            return lax.fori_loop(0, size // 16, svec, jnp.int32(0))

        def do_chunk(chbase, size, cursor):
            cnt = scan_chunk(chbase, size)

            @pl.when(cnt > 0)
            def _():
                last_id = jnp.full((16,), idsbuf[pl.ds(cnt - 1, 16)][0], jnp.int32)
                last_d = jnp.full((16,), dstbuf[pl.ds(cnt - 1, 16)][0], jnp.int32)
                posp = jnp.full((16,), cnt, jnp.int32) + iota16
                ones = jnp.full((16,), True, jnp.bool_)
                plsc.store_scatter(idsbuf, [posp], last_id, mask=ones)
                plsc.store_scatter(dstbuf, [posp], last_d, mask=ones)
                off = pl.multiple_of(wid * cap + cursor, 8)
                pltpu.sync_copy(idsbuf, ids_hbm.at[pl.ds(off, ch + 16)])
                pltpu.sync_copy(dstbuf, dsts_hbm.at[pl.ds(off, ch + 16)])

            cnt8 = (cnt + 7) & ~7
            return cursor + cnt8

        def chunk_body(i, cursor):
            return do_chunk(i * ch, ch, cursor)

        cursor = lax.fori_loop(0, full, chunk_body, jnp.int32(0))
        if tail:
            cursor = do_chunk(full * ch, tail, cursor)

        def zeroids(i, _):
            idsbuf[pl.ds(i * 16, 16)] = zero16
            dstbuf[pl.ds(i * 16, 16)] = lov
            return 0

        lax.fori_loop(0, (ch + 16) // 16, zeroids, 0)
        off = pl.multiple_of(wid * cap + cursor, 8)
        pltpu.sync_copy(idsbuf, ids_hbm.at[pl.ds(off, ch + 16)])
        pltpu.sync_copy(dstbuf, dsts_hbm.at[pl.ds(off, ch + 16)])
        cbuf[pl.ds(0, 16)] = jnp.full((16,), cursor, jnp.int32)
        pltpu.sync_copy(cbuf, cnts_hbm.at[pl.ds(wid * 16, 16)])

    return k(dst)


def _sc_scatter_apply(m, ids, dsts, cnts, n_pad, cap):
    rows_per_tile = n_pad // _NS
    gc = 128

    @functools.partial(
        pl.kernel,
        out_type=jax.ShapeDtypeStruct((_NC * n_pad * 128,), jnp.float32),
        mesh=_sc_mesh(),
        compiler_params=pltpu.CompilerParams(needs_layout_passes=False),
        scratch_types=[
            pltpu.VMEM((gc,), jnp.int32),
            pltpu.VMEM((gc + 16,), jnp.int32),
            pltpu.VMEM((gc,), jnp.int32),
            pltpu.VMEM((gc + 16,), jnp.int32),
            pltpu.VMEM((16,), jnp.int32),
            pltpu.VMEM((gc, 128), jnp.float32),
            pltpu.VMEM((gc, 128), jnp.float32),
            pltpu.VMEM((rows_per_tile * 128,), jnp.float32),
            pltpu.SemaphoreType.DMA,
            pltpu.SemaphoreType.DMA,
        ],
    )
    def k(m_hbm, ids_hbm, dsts_hbm, cnts_hbm, out_hbm,
          gidx0, dstc0, gidx1, dstc1, cbuf, rows0, rows1, acc, sem0, sem1):
        c = lax.axis_index("c")
        s = lax.axis_index("s")
        wid = s * _NC + c
        lo = s * rows_per_tile

        neg = jnp.full((16,), _NEG_INF, jnp.float32)

        def initacc(i, _):
            acc[pl.ds(i * 16, 16)] = neg
            return 0

        lax.fori_loop(0, rows_per_tile * 128 // 16, initacc, 0)

        pltpu.sync_copy(cnts_hbm.at[pl.ds(wid * 16, 16)], cbuf)
        cnt = cbuf[pl.ds(0, 16)][0]
        nsub = (cnt + gc - 1) // gc

        def start(k2, gidx, dstc, sem, rows):
            sbase = k2 * gc
            soff = pl.multiple_of(wid * cap + sbase, 8)
            pltpu.sync_copy(ids_hbm.at[pl.ds(soff, gc)], gidx)
            pltpu.sync_copy(dsts_hbm.at[pl.ds(soff, gc)], dstc.at[pl.ds(0, gc)])
            return pltpu.async_copy(m_hbm.at[gidx], rows, sem)

        def apply_chunk(k2, dstc, rows):
            napply = jnp.minimum(cnt - k2 * gc, gc)

            def apply(r, _):
                dv = dstc[pl.ds(r, 16)][0]
                ab = (dv - lo) * 128
                for j2 in range(8):
                    a = acc[pl.ds(ab + j2 * 16, 16)]
                    b = rows[r, pl.ds(j2 * 16, 16)]
                    acc[pl.ds(ab + j2 * 16, 16)] = jnp.maximum(a, b)
                return 0

            lax.fori_loop(0, napply, apply, 0)

        def pair(j, _):
            g0 = start(2 * j, gidx0, dstc0, sem0, rows0)
            g1 = start(2 * j + 1, gidx1, dstc1, sem1, rows1)
            g0.wait()
            apply_chunk(2 * j, dstc0, rows0)
            g1.wait()
            apply_chunk(2 * j + 1, dstc1, rows1)
            return 0

        lax.fori_loop(0, nsub // 2, pair, 0)

        @pl.when(nsub % 2 == 1)
        def _():
            k2 = nsub - 1
            start(k2, gidx0, dstc0, sem0, rows0).wait()
            apply_chunk(k2, dstc0, rows0)

        obase = (c * n_pad + lo) * 128
        pltpu.sync_copy(acc, out_hbm.at[pl.ds(obase, rows_per_tile * 128)])

    return k(m, ids, dsts, cnts)


def _sc_scatter_max(m, dst, n_pad):
    ids, dsts, cnts = _sc_scatter_prep(dst, n_pad)
    cap = ids.shape[0] // _NW
    return _sc_scatter_apply(m, ids, dsts, cnts, n_pad, cap)


def kernel(h, edge_index, edge_features, W1a, b1a, W2a, b2a, W1b, b1b, W2b, b2b, Wr, br):
    n = h.shape[0]
    n_pad = (n + _NS - 1) // _NS * _NS

    src1 = edge_index[0]
    dst1 = edge_index[1]
    src2 = edge_index[2]
    dst2 = edge_index[3]
    ef0 = edge_features[0::2]
    ef1 = edge_features[1::2]

    pd1, ps1 = _prep1(h, W1a[:128], W1a[128:256])
    u1 = _sc_gather_add(pd1, ps1, dst1, src1)
    m1 = _mlp(u1, ef0, W1a[256:], b1a, W2a, b2a)
    part1 = _sc_scatter_max(m1, dst1, n_pad).reshape(_NC, n_pad, 128)

    pd2, ps2 = _prep2(part1[0], part1[1], W1b[:128], W1b[128:256])
    u2 = _sc_gather_add(pd2, ps2, dst2, src2)
    m2 = _mlp(u2, ef1, W1b[256:], b1b, W2b, b2b)
    part2 = _sc_scatter_max(m2, dst2, n_pad).reshape(_NC, n_pad, 128)

    sel0 = part2[0, 8 : n - 1 : 9]
    sel1 = part2[1, 8 : n - 1 : 9]
    rows = sel0.shape[0]
    rows_pad = (rows + 7) // 8 * 8
    pad = rows_pad - rows
    sel0 = jnp.pad(sel0, ((0, pad), (0, 0)))
    sel1 = jnp.pad(sel1, ((0, pad), (0, 0)))
    o = _final(sel0, sel1, Wr, br)
    return o[:rows]


# speedup vs baseline: 4.5069x; 1.0083x over previous
import functools

import jax
import jax.numpy as jnp
from jax import lax
from jax.experimental import pallas as pl
from jax.experimental.pallas import tpu as pltpu
from jax.experimental.pallas import tpu_sc as plsc

_NC = 2
_NS = 16
_NW = _NC * _NS

_NEG_INF = float("-inf")


def _sc_mesh():
    return plsc.VectorSubcoreMesh(core_axis_name="c", subcore_axis_name="s")


def _prep1_body(h_ref, wd_ref, ws_ref, pd_ref, ps_ref):
    hb = h_ref[...]
    pd_ref[...] = jnp.dot(hb, wd_ref[...], preferred_element_type=jnp.float32)
    ps_ref[...] = jnp.dot(hb, ws_ref[...], preferred_element_type=jnp.float32)


def _prep1(h, wd, ws):
    n = h.shape[0]
    blk = 2000
    assert n % blk == 0
    return pl.pallas_call(
        _prep1_body,
        grid=(n // blk,),
        in_specs=[
            pl.BlockSpec((blk, 128), lambda i: (i, 0)),
            pl.BlockSpec((128, 128), lambda i: (0, 0)),
            pl.BlockSpec((128, 128), lambda i: (0, 0)),
        ],
        out_specs=[
            pl.BlockSpec((blk, 128), lambda i: (i, 0)),
            pl.BlockSpec((blk, 128), lambda i: (i, 0)),
        ],
        out_shape=[
            jax.ShapeDtypeStruct((n, 128), jnp.float32),
            jax.ShapeDtypeStruct((n, 128), jnp.float32),
        ],
    )(h, wd, ws)


def _prep2_body(p0_ref, p1_ref, wd_ref, ws_ref, pd_ref, ps_ref):
    hm = jnp.maximum(p0_ref[...], p1_ref[...])
    hm = jnp.where(hm == _NEG_INF, 0.0, hm)
    pd_ref[...] = jnp.dot(hm, wd_ref[...], preferred_element_type=jnp.float32)
    ps_ref[...] = jnp.dot(hm, ws_ref[...], preferred_element_type=jnp.float32)


def _prep2(p0, p1, wd, ws):
    n = p0.shape[0]
    blk = 2000
    assert n % blk == 0
    return pl.pallas_call(
        _prep2_body,
        grid=(n // blk,),
        in_specs=[
            pl.BlockSpec((blk, 128), lambda i: (i, 0)),
            pl.BlockSpec((blk, 128), lambda i: (i, 0)),
            pl.BlockSpec((128, 128), lambda i: (0, 0)),
            pl.BlockSpec((128, 128), lambda i: (0, 0)),
        ],
        out_specs=[
            pl.BlockSpec((blk, 128), lambda i: (i, 0)),
            pl.BlockSpec((blk, 128), lambda i: (i, 0)),
        ],
        out_shape=[
            jax.ShapeDtypeStruct((n, 128), jnp.float32),
            jax.ShapeDtypeStruct((n, 128), jnp.float32),
        ],
    )(p0, p1, wd, ws)


def _mlp_body(u_ref, ef_ref, wef_ref, b1_ref, w2_ref, b2_ref, out_ref):
    ef = ef_ref[...]
    wef = wef_ref[...]
    x = u_ref[...] + b1_ref[...]
    x = x + ef[:, 0:1] * wef[0:1, :]
    x = x + ef[:, 1:2] * wef[1:2, :]
    x = x + ef[:, 2:3] * wef[2:3, :]
    x = jnp.maximum(x, 0.0)
    out_ref[...] = (
        jnp.dot(x, w2_ref[...], preferred_element_type=jnp.float32) + b2_ref[...]
    )


def _mlp(u, ef, wef, b1, w2, b2):
    e = u.shape[0]
    blk = 2000
    assert e % blk == 0
    return pl.pallas_call(
        _mlp_body,
        grid=(e // blk,),
        in_specs=[
            pl.BlockSpec((blk, 128), lambda i: (i, 0)),
            pl.BlockSpec((blk, 3), lambda i: (i, 0)),
            pl.BlockSpec((3, 128), lambda i: (0, 0)),
            pl.BlockSpec((1, 128), lambda i: (0, 0)),
            pl.BlockSpec((128, 128), lambda i: (0, 0)),
            pl.BlockSpec((1, 128), lambda i: (0, 0)),
        ],
        out_specs=pl.BlockSpec((blk, 128), lambda i: (i, 0)),
        out_shape=jax.ShapeDtypeStruct((e, 128), jnp.float32),
    )(u, ef, wef, b1.reshape(1, 128), w2, b2.reshape(1, 128))


def _final_body(p0_ref, p1_ref, wr_ref, br_ref, o_ref):
    hm = jnp.maximum(p0_ref[...], p1_ref[...])
    hm = jnp.where(hm == _NEG_INF, 0.0, hm)
    o_ref[...] = (
        jnp.dot(hm, wr_ref[...], preferred_element_type=jnp.float32) + br_ref[...]
    )


def _final(p0, p1, wr, br):
    n = p0.shape[0]
    return pl.pallas_call(
        _final_body,
        out_shape=jax.ShapeDtypeStruct((n, 3), jnp.float32),
    )(p0, p1, wr, br.reshape(1, 3))


def _sc_gather_add(pd, ps, dst, src):
    n_edges = dst.shape[0]
    assert n_edges % _NW == 0
    per_w = n_edges // _NW
    c_sz = 128
    full = per_w // c_sz
    tail = per_w % c_sz
    assert tail % 8 == 0

    assert full % 3 == 0

    @functools.partial(
        pl.kernel,
        out_type=jax.ShapeDtypeStruct((n_edges, 128), jnp.float32),
        mesh=_sc_mesh(),
        compiler_params=pltpu.CompilerParams(needs_layout_passes=False),
        scratch_types=[
            pltpu.VMEM((c_sz,), jnp.int32),
            pltpu.VMEM((c_sz,), jnp.int32),
            pltpu.VMEM((c_sz, 128), jnp.float32),
            pltpu.VMEM((c_sz,), jnp.int32),
            pltpu.VMEM((c_sz,), jnp.int32),
            pltpu.VMEM((c_sz, 128), jnp.float32),
            pltpu.VMEM((c_sz,), jnp.int32),
            pltpu.VMEM((c_sz,), jnp.int32),
            pltpu.VMEM((c_sz, 128), jnp.float32),
            pltpu.SemaphoreType.DMA,
            pltpu.SemaphoreType.DMA,
            pltpu.SemaphoreType.DMA,
        ],
    )
    def k(pd_hbm, ps_hbm, dst_hbm, src_hbm, u_hbm,
          didx0, sidx0, rows0, didx1, sidx1, rows1, didx2, sidx2, rows2,
          sem0, sem1, sem2):
        wid = lax.axis_index("s") * _NC + lax.axis_index("c")
        base = wid * per_w

        def triple(i, _):
            off0 = base + (3 * i) * c_sz
            off1 = off0 + c_sz
            off2 = off1 + c_sz
            pltpu.sync_copy(dst_hbm.at[pl.ds(off0, c_sz)], didx0)
            pltpu.sync_copy(src_hbm.at[pl.ds(off0, c_sz)], sidx0)
            g0 = pltpu.async_copy(pd_hbm.at[didx0], rows0, sem0)
            pltpu.sync_copy(dst_hbm.at[pl.ds(off1, c_sz)], didx1)
            pltpu.sync_copy(src_hbm.at[pl.ds(off1, c_sz)], sidx1)
            g1 = pltpu.async_copy(pd_hbm.at[didx1], rows1, sem1)
            pltpu.sync_copy(dst_hbm.at[pl.ds(off2, c_sz)], didx2)
            pltpu.sync_copy(src_hbm.at[pl.ds(off2, c_sz)], sidx2)
            g2 = pltpu.async_copy(pd_hbm.at[didx2], rows2, sem2)
            g0.wait()
            a0 = pltpu.async_copy(ps_hbm.at[sidx0], rows0, sem0, add=True)
            g1.wait()
            a1 = pltpu.async_copy(ps_hbm.at[sidx1], rows1, sem1, add=True)
            g2.wait()
            a2 = pltpu.async_copy(ps_hbm.at[sidx2], rows2, sem2, add=True)
            a0.wait()
            pltpu.sync_copy(rows0, u_hbm.at[pl.ds(off0, c_sz)])
            a1.wait()
            pltpu.sync_copy(rows1, u_hbm.at[pl.ds(off1, c_sz)])
            a2.wait()
            pltpu.sync_copy(rows2, u_hbm.at[pl.ds(off2, c_sz)])
            return 0

        lax.fori_loop(0, full // 3, triple, 0)
        if tail:
            off = base + full * c_sz
            dbuf = didx0.at[pl.ds(0, tail)]
            sbuf = sidx0.at[pl.ds(0, tail)]
            rbuf = rows0.at[pl.ds(0, tail)]
            pltpu.sync_copy(dst_hbm.at[pl.ds(off, tail)], dbuf)
            pltpu.sync_copy(src_hbm.at[pl.ds(off, tail)], sbuf)
            pltpu.async_copy(pd_hbm.at[dbuf], rbuf, sem0).wait()
            pltpu.async_copy(ps_hbm.at[sbuf], rbuf, sem0, add=True).wait()
            pltpu.sync_copy(rbuf, u_hbm.at[pl.ds(off, tail)])

    return k(pd, ps, dst, src)


def _sc_scatter_prep(dst, n_pad):
    n_edges = dst.shape[0]
    assert n_edges % _NC == 0
    half = n_edges // _NC
    rows_per_tile = n_pad // _NS
    ch = 2048
    full = half // ch
    tail = half % ch
    assert tail % 16 == 0
    cap = half + 2 * (ch + 16)
    assert cap % 8 == 0

    @functools.partial(
        pl.kernel,
        out_type=(
            jax.ShapeDtypeStruct((_NW * cap,), jnp.int32),
            jax.ShapeDtypeStruct((_NW * cap,), jnp.int32),
            jax.ShapeDtypeStruct((_NW * 16,), jnp.int32),
        ),
        mesh=_sc_mesh(),
        compiler_params=pltpu.CompilerParams(needs_layout_passes=False),
        scratch_types=[
            pltpu.VMEM((ch,), jnp.int32),
            pltpu.VMEM((ch + 16,), jnp.int32),
            pltpu.VMEM((ch + 16,), jnp.int32),
            pltpu.VMEM((16,), jnp.int32),
        ],
    )
    def k(dst_hbm, ids_hbm, dsts_hbm, cnts_hbm, dbuf, idsbuf, dstbuf, cbuf):
        c = lax.axis_index("c")
        s = lax.axis_index("s")
        wid = s * _NC + c
        lo = s * rows_per_tile
        hi = lo + rows_per_tile
        ebase = c * half
        iota16 = lax.iota(jnp.int32, 16)
        lov = jnp.full((16,), lo, jnp.int32)
        hiv = jnp.full((16,), hi, jnp.int32)
        zero16 = jnp.zeros((16,), jnp.int32)

        def initids(i, _):
            idsbuf[pl.ds(i * 16, 16)] = zero16
            dstbuf[pl.ds(i * 16, 16)] = lov
            return 0

        lax.fori_loop(0, (ch + 16) // 16, initids, 0)

        def scan_chunk(chbase, size):
            pltpu.sync_copy(
                dst_hbm.at[pl.ds(ebase + chbase, size)], dbuf.at[pl.ds(0, size)]
            )

            def svec(v, cnt):
                d = dbuf[pl.ds(v * 16, 16)]
                eid = jnp.full((16,), ebase + chbase + v * 16, jnp.int32) + iota16
                mask = (d >= lov) & (d < hiv)
                cs = plsc.cumsum(mask.astype(jnp.int32))
                pos = jnp.full((16,), cnt - 1, jnp.int32) + cs
                plsc.store_scatter(idsbuf, [pos], eid, mask=mask)
                plsc.store_scatter(dstbuf, [pos], d, mask=mask)
                return cnt + cs[15]

            return lax.fori_loop(0, size // 16, svec, jnp.int32(0))

        def do_chunk(chbase, size, cursor):
            cnt = scan_chunk(chbase, size)

            @pl.when(cnt > 0)
            def _():
                last_id = jnp.full((16,), idsbuf[pl.ds(cnt - 1, 16)][0], jnp.int32)
                last_d = jnp.full((16,), dstbuf[pl.ds(cnt - 1, 16)][0], jnp.int32)
                posp = jnp.full((16,), cnt, jnp.int32) + iota16
                ones = jnp.full((16,), True, jnp.bool_)
                plsc.store_scatter(idsbuf, [posp], last_id, mask=ones)
                plsc.store_scatter(dstbuf, [posp], last_d, mask=ones)
                off = pl.multiple_of(wid * cap + cursor, 8)
                pltpu.sync_copy(idsbuf, ids_hbm.at[pl.ds(off, ch + 16)])
                pltpu.sync_copy(dstbuf, dsts_hbm.at[pl.ds(off, ch + 16)])

            cnt8 = (cnt + 7) & ~7
            return cursor + cnt8

        def chunk_body(i, cursor):
            return do_chunk(i * ch, ch, cursor)

        cursor = lax.fori_loop(0, full, chunk_body, jnp.int32(0))
        if tail:
            cursor = do_chunk(full * ch, tail, cursor)

        def zeroids(i, _):
            idsbuf[pl.ds(i * 16, 16)] = zero16
            dstbuf[pl.ds(i * 16, 16)] = lov
            return 0

        lax.fori_loop(0, (ch + 16) // 16, zeroids, 0)
        off = pl.multiple_of(wid * cap + cursor, 8)
        pltpu.sync_copy(idsbuf, ids_hbm.at[pl.ds(off, ch + 16)])
        pltpu.sync_copy(dstbuf, dsts_hbm.at[pl.ds(off, ch + 16)])
        cbuf[pl.ds(0, 16)] = jnp.full((16,), cursor, jnp.int32)
        pltpu.sync_copy(cbuf, cnts_hbm.at[pl.ds(wid * 16, 16)])

    return k(dst)


def _sc_scatter_apply(m, ids, dsts, cnts, n_pad, cap):
    rows_per_tile = n_pad // _NS
    gc = 128

    @functools.partial(
        pl.kernel,
        out_type=jax.ShapeDtypeStruct((_NC * n_pad * 128,), jnp.float32),
        mesh=_sc_mesh(),
        compiler_params=pltpu.CompilerParams(needs_layout_passes=False),
        scratch_types=[
            pltpu.VMEM((gc,), jnp.int32),
            pltpu.VMEM((gc + 16,), jnp.int32),
            pltpu.VMEM((gc,), jnp.int32),
            pltpu.VMEM((gc + 16,), jnp.int32),
            pltpu.VMEM((16,), jnp.int32),
            pltpu.VMEM((gc, 128), jnp.float32),
            pltpu.VMEM((gc, 128), jnp.float32),
            pltpu.VMEM((rows_per_tile * 128,), jnp.float32),
            pltpu.SemaphoreType.DMA,
            pltpu.SemaphoreType.DMA,
        ],
    )
    def k(m_hbm, ids_hbm, dsts_hbm, cnts_hbm, out_hbm,
          gidx0, dstc0, gidx1, dstc1, cbuf, rows0, rows1, acc, sem0, sem1):
        c = lax.axis_index("c")
        s = lax.axis_index("s")
        wid = s * _NC + c
        lo = s * rows_per_tile

        neg = jnp.full((16,), _NEG_INF, jnp.float32)

        def initacc(i, _):
            acc[pl.ds(i * 16, 16)] = neg
            return 0

        lax.fori_loop(0, rows_per_tile * 128 // 16, initacc, 0)

        pltpu.sync_copy(cnts_hbm.at[pl.ds(wid * 16, 16)], cbuf)
        cnt = cbuf[pl.ds(0, 16)][0]
        nsub = (cnt + gc - 1) // gc

        def start(k2, gidx, dstc, sem, rows):
            sbase = k2 * gc
            soff = pl.multiple_of(wid * cap + sbase, 8)
            pltpu.sync_copy(ids_hbm.at[pl.ds(soff, gc)], gidx)
            pltpu.sync_copy(dsts_hbm.at[pl.ds(soff, gc)], dstc.at[pl.ds(0, gc)])
            return pltpu.async_copy(m_hbm.at[gidx], rows, sem)

        def apply_chunk(k2, dstc, rows):
            napply = jnp.minimum(cnt - k2 * gc, gc)

            def apply(r, _):
                dv = dstc[pl.ds(r, 16)][0]
                ab = (dv - lo) * 128
                for j2 in range(8):
                    a = acc[pl.ds(ab + j2 * 16, 16)]
                    b = rows[r, pl.ds(j2 * 16, 16)]
                    acc[pl.ds(ab + j2 * 16, 16)] = jnp.maximum(a, b)
                return 0

            lax.fori_loop(0, napply, apply, 0)

        def pair(j, _):
            g0 = start(2 * j, gidx0, dstc0, sem0, rows0)
            g1 = start(2 * j + 1, gidx1, dstc1, sem1, rows1)
            g0.wait()
            apply_chunk(2 * j, dstc0, rows0)
            g1.wait()
            apply_chunk(2 * j + 1, dstc1, rows1)
            return 0

        lax.fori_loop(0, nsub // 2, pair, 0)

        @pl.when(nsub % 2 == 1)
        def _():
            k2 = nsub - 1
            start(k2, gidx0, dstc0, sem0, rows0).wait()
            apply_chunk(k2, dstc0, rows0)

        obase = (c * n_pad + lo) * 128
        pltpu.sync_copy(acc, out_hbm.at[pl.ds(obase, rows_per_tile * 128)])

    return k(m, ids, dsts, cnts)


def _sc_scatter_max(m, dst, n_pad):
    ids, dsts, cnts = _sc_scatter_prep(dst, n_pad)
    cap = ids.shape[0] // _NW
    return _sc_scatter_apply(m, ids, dsts, cnts, n_pad, cap)


def kernel(h, edge_index, edge_features, W1a, b1a, W2a, b2a, W1b, b1b, W2b, b2b, Wr, br):
    n = h.shape[0]
    n_pad = (n + _NS - 1) // _NS * _NS

    src1 = edge_index[0]
    dst1 = edge_index[1]
    src2 = edge_index[2]
    dst2 = edge_index[3]
    ef0 = edge_features[0::2]
    ef1 = edge_features[1::2]

    pd1, ps1 = _prep1(h, W1a[:128], W1a[128:256])
    u1 = _sc_gather_add(pd1, ps1, dst1, src1)
    m1 = _mlp(u1, ef0, W1a[256:], b1a, W2a, b2a)
    part1 = _sc_scatter_max(m1, dst1, n_pad).reshape(_NC, n_pad, 128)

    pd2, ps2 = _prep2(part1[0], part1[1], W1b[:128], W1b[128:256])
    u2 = _sc_gather_add(pd2, ps2, dst2, src2)
    m2 = _mlp(u2, ef1, W1b[256:], b1b, W2b, b2b)
    part2 = _sc_scatter_max(m2, dst2, n_pad).reshape(_NC, n_pad, 128)

    sel0 = part2[0, 8 : n - 1 : 9]
    sel1 = part2[1, 8 : n - 1 : 9]
    rows = sel0.shape[0]
    rows_pad = (rows + 7) // 8 * 8
    pad = rows_pad - rows
    sel0 = jnp.pad(sel0, ((0, pad), (0, 0)))
    sel1 = jnp.pad(sel1, ((0, pad), (0, 0)))
    o = _final(sel0, sel1, Wr, br)
    return o[:rows]


# speedup vs baseline: 4.5103x; 1.0008x over previous
import functools

import jax
import jax.numpy as jnp
from jax import lax
from jax.experimental import pallas as pl
from jax.experimental.pallas import tpu as pltpu
from jax.experimental.pallas import tpu_sc as plsc

_NC = 2
_NS = 16
_NW = _NC * _NS

_NEG_INF = float("-inf")


def _sc_mesh():
    return plsc.VectorSubcoreMesh(core_axis_name="c", subcore_axis_name="s")


def _prep1_body(h_ref, wd_ref, ws_ref, pd_ref, ps_ref):
    hb = h_ref[...]
    pd_ref[...] = jnp.dot(hb, wd_ref[...], preferred_element_type=jnp.float32)
    ps_ref[...] = jnp.dot(hb, ws_ref[...], preferred_element_type=jnp.float32)


def _prep1(h, wd, ws):
    n = h.shape[0]
    blk = 2000
    assert n % blk == 0
    return pl.pallas_call(
        _prep1_body,
        grid=(n // blk,),
        in_specs=[
            pl.BlockSpec((blk, 128), lambda i: (i, 0)),
            pl.BlockSpec((128, 128), lambda i: (0, 0)),
            pl.BlockSpec((128, 128), lambda i: (0, 0)),
        ],
        out_specs=[
            pl.BlockSpec((blk, 128), lambda i: (i, 0)),
            pl.BlockSpec((blk, 128), lambda i: (i, 0)),
        ],
        out_shape=[
            jax.ShapeDtypeStruct((n, 128), jnp.float32),
            jax.ShapeDtypeStruct((n, 128), jnp.float32),
        ],
    )(h, wd, ws)


def _prep2_body(p0_ref, p1_ref, wd_ref, ws_ref, pd_ref, ps_ref):
    hm = jnp.maximum(p0_ref[...], p1_ref[...])
    hm = jnp.where(hm == _NEG_INF, 0.0, hm)
    pd_ref[...] = jnp.dot(hm, wd_ref[...], preferred_element_type=jnp.float32)
    ps_ref[...] = jnp.dot(hm, ws_ref[...], preferred_element_type=jnp.float32)


def _prep2(p0, p1, wd, ws):
    n = p0.shape[0]
    blk = 2000
    assert n % blk == 0
    return pl.pallas_call(
        _prep2_body,
        grid=(n // blk,),
        in_specs=[
            pl.BlockSpec((blk, 128), lambda i: (i, 0)),
            pl.BlockSpec((blk, 128), lambda i: (i, 0)),
            pl.BlockSpec((128, 128), lambda i: (0, 0)),
            pl.BlockSpec((128, 128), lambda i: (0, 0)),
        ],
        out_specs=[
            pl.BlockSpec((blk, 128), lambda i: (i, 0)),
            pl.BlockSpec((blk, 128), lambda i: (i, 0)),
        ],
        out_shape=[
            jax.ShapeDtypeStruct((n, 128), jnp.float32),
            jax.ShapeDtypeStruct((n, 128), jnp.float32),
        ],
    )(p0, p1, wd, ws)


def _mlp_body(u_ref, ef_ref, wef_ref, b1_ref, w2_ref, b2_ref, out_ref):
    ef = ef_ref[...]
    wef = wef_ref[...]
    x = u_ref[...] + b1_ref[...]
    x = x + ef[:, 0:1] * wef[0:1, :]
    x = x + ef[:, 1:2] * wef[1:2, :]
    x = x + ef[:, 2:3] * wef[2:3, :]
    x = jnp.maximum(x, 0.0)
    out_ref[...] = (
        jnp.dot(x, w2_ref[...], preferred_element_type=jnp.float32) + b2_ref[...]
    )


def _mlp(u, ef, wef, b1, w2, b2):
    e = u.shape[0]
    blk = 2000
    assert e % blk == 0
    return pl.pallas_call(
        _mlp_body,
        grid=(e // blk,),
        in_specs=[
            pl.BlockSpec((blk, 128), lambda i: (i, 0)),
            pl.BlockSpec((blk, 3), lambda i: (i, 0)),
            pl.BlockSpec((3, 128), lambda i: (0, 0)),
            pl.BlockSpec((1, 128), lambda i: (0, 0)),
            pl.BlockSpec((128, 128), lambda i: (0, 0)),
            pl.BlockSpec((1, 128), lambda i: (0, 0)),
        ],
        out_specs=pl.BlockSpec((blk, 128), lambda i: (i, 0)),
        out_shape=jax.ShapeDtypeStruct((e, 128), jnp.float32),
    )(u, ef, wef, b1.reshape(1, 128), w2, b2.reshape(1, 128))


def _final_body(p0_ref, p1_ref, wr_ref, br_ref, o_ref):
    hm = jnp.maximum(p0_ref[...], p1_ref[...])
    hm = jnp.where(hm == _NEG_INF, 0.0, hm)
    o_ref[...] = (
        jnp.dot(hm, wr_ref[...], preferred_element_type=jnp.float32) + br_ref[...]
    )


def _final(p0, p1, wr, br):
    n = p0.shape[0]
    return pl.pallas_call(
        _final_body,
        out_shape=jax.ShapeDtypeStruct((n, 3), jnp.float32),
    )(p0, p1, wr, br.reshape(1, 3))


def _sc_gather_add(pd, ps, dst, src):
    n_edges = dst.shape[0]
    assert n_edges % _NW == 0
    per_w = n_edges // _NW
    c_sz = 128
    full = per_w // c_sz
    tail = per_w % c_sz
    assert tail % 8 == 0

    assert full % 3 == 0

    @functools.partial(
        pl.kernel,
        out_type=jax.ShapeDtypeStruct((n_edges, 128), jnp.float32),
        mesh=_sc_mesh(),
        compiler_params=pltpu.CompilerParams(needs_layout_passes=False),
        scratch_types=[
            pltpu.VMEM((c_sz,), jnp.int32),
            pltpu.VMEM((c_sz,), jnp.int32),
            pltpu.VMEM((c_sz, 128), jnp.float32),
            pltpu.VMEM((c_sz,), jnp.int32),
            pltpu.VMEM((c_sz,), jnp.int32),
            pltpu.VMEM((c_sz, 128), jnp.float32),
            pltpu.VMEM((c_sz,), jnp.int32),
            pltpu.VMEM((c_sz,), jnp.int32),
            pltpu.VMEM((c_sz, 128), jnp.float32),
            pltpu.SemaphoreType.DMA,
            pltpu.SemaphoreType.DMA,
            pltpu.SemaphoreType.DMA,
        ],
    )
    def k(pd_hbm, ps_hbm, dst_hbm, src_hbm, u_hbm,
          didx0, sidx0, rows0, didx1, sidx1, rows1, didx2, sidx2, rows2,
          sem0, sem1, sem2):
        wid = lax.axis_index("s") * _NC + lax.axis_index("c")
        base = wid * per_w

        def triple(i, _):
            off0 = base + (3 * i) * c_sz
            off1 = off0 + c_sz
            off2 = off1 + c_sz
            pltpu.sync_copy(dst_hbm.at[pl.ds(off0, c_sz)], didx0)
            pltpu.sync_copy(src_hbm.at[pl.ds(off0, c_sz)], sidx0)
            g0 = pltpu.async_copy(pd_hbm.at[didx0], rows0, sem0)
            pltpu.sync_copy(dst_hbm.at[pl.ds(off1, c_sz)], didx1)
            pltpu.sync_copy(src_hbm.at[pl.ds(off1, c_sz)], sidx1)
            g1 = pltpu.async_copy(pd_hbm.at[didx1], rows1, sem1)
            pltpu.sync_copy(dst_hbm.at[pl.ds(off2, c_sz)], didx2)
            pltpu.sync_copy(src_hbm.at[pl.ds(off2, c_sz)], sidx2)
            g2 = pltpu.async_copy(pd_hbm.at[didx2], rows2, sem2)
            g0.wait()
            a0 = pltpu.async_copy(ps_hbm.at[sidx0], rows0, sem0, add=True)
            g1.wait()
            a1 = pltpu.async_copy(ps_hbm.at[sidx1], rows1, sem1, add=True)
            g2.wait()
            a2 = pltpu.async_copy(ps_hbm.at[sidx2], rows2, sem2, add=True)
            a0.wait()
            pltpu.sync_copy(rows0, u_hbm.at[pl.ds(off0, c_sz)])
            a1.wait()
            pltpu.sync_copy(rows1, u_hbm.at[pl.ds(off1, c_sz)])
            a2.wait()
            pltpu.sync_copy(rows2, u_hbm.at[pl.ds(off2, c_sz)])
            return 0

        lax.fori_loop(0, full // 3, triple, 0)
        if tail:
            off = base + full * c_sz
            dbuf = didx0.at[pl.ds(0, tail)]
            sbuf = sidx0.at[pl.ds(0, tail)]
            rbuf = rows0.at[pl.ds(0, tail)]
            pltpu.sync_copy(dst_hbm.at[pl.ds(off, tail)], dbuf)
            pltpu.sync_copy(src_hbm.at[pl.ds(off, tail)], sbuf)
            pltpu.async_copy(pd_hbm.at[dbuf], rbuf, sem0).wait()
            pltpu.async_copy(ps_hbm.at[sbuf], rbuf, sem0, add=True).wait()
            pltpu.sync_copy(rbuf, u_hbm.at[pl.ds(off, tail)])

    return k(pd, ps, dst, src)


def _sc_scatter_prep(dst, n_pad):
    n_edges = dst.shape[0]
    assert n_edges % _NC == 0
    half = n_edges // _NC
    rows_per_tile = n_pad // _NS
    ch = 2048
    full = half // ch
    tail = half % ch
    assert tail % 16 == 0
    cap = half + 2 * (ch + 16)
    assert cap % 8 == 0

    @functools.partial(
        pl.kernel,
        out_type=(
            jax.ShapeDtypeStruct((_NW * cap,), jnp.int32),
            jax.ShapeDtypeStruct((_NW * cap,), jnp.int32),
            jax.ShapeDtypeStruct((_NW * 16,), jnp.int32),
        ),
        mesh=_sc_mesh(),
        compiler_params=pltpu.CompilerParams(needs_layout_passes=False),
        scratch_types=[
            pltpu.VMEM((ch,), jnp.int32),
            pltpu.VMEM((ch + 16,), jnp.int32),
            pltpu.VMEM((ch + 16,), jnp.int32),
            pltpu.VMEM((16,), jnp.int32),
        ],
    )
    def k(dst_hbm, ids_hbm, dsts_hbm, cnts_hbm, dbuf, idsbuf, dstbuf, cbuf):
        c = lax.axis_index("c")
        s = lax.axis_index("s")
        wid = s * _NC + c
        lo = s * rows_per_tile
        hi = lo + rows_per_tile
        ebase = c * half
        iota16 = lax.iota(jnp.int32, 16)
        lov = jnp.full((16,), lo, jnp.int32)
        hiv = jnp.full((16,), hi, jnp.int32)
        zero16 = jnp.zeros((16,), jnp.int32)

        def initids(i, _):
            idsbuf[pl.ds(i * 16, 16)] = zero16
            dstbuf[pl.ds(i * 16, 16)] = lov
            return 0

        lax.fori_loop(0, (ch + 16) // 16, initids, 0)

        def scan_chunk(chbase, size):
            pltpu.sync_copy(
                dst_hbm.at[pl.ds(ebase + chbase, size)], dbuf.at[pl.ds(0, size)]
            )

            def svec(v, cnt):
                d = dbuf[pl.ds(v * 16, 16)]
                eid = jnp.full((16,), ebase + chbase + v * 16, jnp.int32) + iota16
                mask = (d >= lov) & (d < hiv)
                cs = plsc.cumsum(mask.astype(jnp.int32))
                pos = jnp.full((16,), cnt - 1, jnp.int32) + cs
                plsc.store_scatter(idsbuf, [pos], eid, mask=mask)
                plsc.store_scatter(dstbuf, [pos], d, mask=mask)
                return cnt + cs[15]

            return lax.fori_loop(0, size // 16, svec, jnp.int32(0))

        def do_chunk(chbase, size, cursor):
            cnt = scan_chunk(chbase, size)

            @pl.when(cnt > 0)
            def _():
                last_id = jnp.full((16,), idsbuf[pl.ds(cnt - 1, 16)][0], jnp.int32)
                last_d = jnp.full((16,), dstbuf[pl.ds(cnt - 1, 16)][0], jnp.int32)
                posp = jnp.full((16,), cnt, jnp.int32) + iota16
                ones = jnp.full((16,), True, jnp.bool_)
                plsc.store_scatter(idsbuf, [posp], last_id, mask=ones)
                plsc.store_scatter(dstbuf, [posp], last_d, mask=ones)
                off = pl.multiple_of(wid * cap + cursor, 8)
                pltpu.sync_copy(idsbuf, ids_hbm.at[pl.ds(off, ch + 16)])
                pltpu.sync_copy(dstbuf, dsts_hbm.at[pl.ds(off, ch + 16)])

            cnt8 = (cnt + 7) & ~7
            return cursor + cnt8

        def chunk_body(i, cursor):
            return do_chunk(i * ch, ch, cursor)

        cursor = lax.fori_loop(0, full, chunk_body, jnp.int32(0))
        if tail:
            cursor = do_chunk(full * ch, tail, cursor)

        def zeroids(i, _):
            idsbuf[pl.ds(i * 16, 16)] = zero16
            dstbuf[pl.ds(i * 16, 16)] = lov
            return 0

        lax.fori_loop(0, (ch + 16) // 16, zeroids, 0)
        off = pl.multiple_of(wid * cap + cursor, 8)
        pltpu.sync_copy(idsbuf, ids_hbm.at[pl.ds(off, ch + 16)])
        pltpu.sync_copy(dstbuf, dsts_hbm.at[pl.ds(off, ch + 16)])
        cbuf[pl.ds(0, 16)] = jnp.full((16,), cursor, jnp.int32)
        pltpu.sync_copy(cbuf, cnts_hbm.at[pl.ds(wid * 16, 16)])

    return k(dst)


def _sc_scatter_apply(m, ids, dsts, cnts, n_pad, cap):
    rows_per_tile = n_pad // _NS
    gc = 128

    @functools.partial(
        pl.kernel,
        out_type=jax.ShapeDtypeStruct((_NC * n_pad * 128,), jnp.float32),
        mesh=_sc_mesh(),
        compiler_params=pltpu.CompilerParams(needs_layout_passes=False),
        scratch_types=[
            pltpu.VMEM((gc,), jnp.int32),
            pltpu.VMEM((gc + 16,), jnp.int32),
            pltpu.VMEM((gc,), jnp.int32),
            pltpu.VMEM((gc + 16,), jnp.int32),
            pltpu.VMEM((16,), jnp.int32),
            pltpu.VMEM((gc, 128), jnp.float32),
            pltpu.VMEM((gc, 128), jnp.float32),
            pltpu.VMEM((rows_per_tile * 128,), jnp.float32),
            pltpu.SemaphoreType.DMA,
            pltpu.SemaphoreType.DMA,
        ],
    )
    def k(m_hbm, ids_hbm, dsts_hbm, cnts_hbm, out_hbm,
          gidx0, dstc0, gidx1, dstc1, cbuf, rows0, rows1, acc, sem0, sem1):
        c = lax.axis_index("c")
        s = lax.axis_index("s")
        wid = s * _NC + c
        lo = s * rows_per_tile

        neg = jnp.full((16,), _NEG_INF, jnp.float32)

        def initacc(i, _):
            acc[pl.ds(i * 16, 16)] = neg
            return 0

        lax.fori_loop(0, rows_per_tile * 128 // 16, initacc, 0)

        pltpu.sync_copy(cnts_hbm.at[pl.ds(wid * 16, 16)], cbuf)
        cnt = cbuf[pl.ds(0, 16)][0]
        nsub = (cnt + gc - 1) // gc

        def start(k2, gidx, dstc, sem, rows):
            sbase = k2 * gc
            soff = pl.multiple_of(wid * cap + sbase, 8)
            pltpu.sync_copy(ids_hbm.at[pl.ds(soff, gc)], gidx)
            pltpu.sync_copy(dsts_hbm.at[pl.ds(soff, gc)], dstc.at[pl.ds(0, gc)])
            return pltpu.async_copy(m_hbm.at[gidx], rows, sem)

        def apply_chunk(k2, dstc, rows):
            napply = jnp.minimum(cnt - k2 * gc, gc)

            def apply(r, _):
                dv = dstc[pl.ds(r, 16)][0]
                ab = (dv - lo) * 128
                for j2 in range(8):
                    a = acc[pl.ds(ab + j2 * 16, 16)]
                    b = rows[r, pl.ds(j2 * 16, 16)]
                    acc[pl.ds(ab + j2 * 16, 16)] = jnp.maximum(a, b)
                return 0

            lax.fori_loop(0, napply, apply, 0)

        def pair(j, _):
            g0 = start(2 * j, gidx0, dstc0, sem0, rows0)
            g1 = start(2 * j + 1, gidx1, dstc1, sem1, rows1)
            g0.wait()
            apply_chunk(2 * j, dstc0, rows0)
            g1.wait()
            apply_chunk(2 * j + 1, dstc1, rows1)
            return 0

        lax.fori_loop(0, nsub // 2, pair, 0)

        @pl.when(nsub % 2 == 1)
        def _():
            k2 = nsub - 1
            start(k2, gidx0, dstc0, sem0, rows0).wait()
            apply_chunk(k2, dstc0, rows0)

        obase = (c * n_pad + lo) * 128
        pltpu.sync_copy(acc, out_hbm.at[pl.ds(obase, rows_per_tile * 128)])

    return k(m, ids, dsts, cnts)


def _sc_scatter_max(m, prep, n_pad):
    ids, dsts, cnts = prep
    cap = ids.shape[0] // _NW
    return _sc_scatter_apply(m, ids, dsts, cnts, n_pad, cap)


def kernel(h, edge_index, edge_features, W1a, b1a, W2a, b2a, W1b, b1b, W2b, b2b, Wr, br):
    n = h.shape[0]
    n_pad = (n + _NS - 1) // _NS * _NS

    src1 = edge_index[0]
    dst1 = edge_index[1]
    src2 = edge_index[2]
    dst2 = edge_index[3]
    ef0 = edge_features[0::2]
    ef1 = edge_features[1::2]

    pd1, ps1 = _prep1(h, W1a[:128], W1a[128:256])
    u1 = _sc_gather_add(pd1, ps1, dst1, src1)
    prep_a = _sc_scatter_prep(dst1, n_pad)
    m1 = _mlp(u1, ef0, W1a[256:], b1a, W2a, b2a)
    part1 = _sc_scatter_max(m1, prep_a, n_pad).reshape(_NC, n_pad, 128)

    pd2, ps2 = _prep2(part1[0], part1[1], W1b[:128], W1b[128:256])
    u2 = _sc_gather_add(pd2, ps2, dst2, src2)
    prep_b = _sc_scatter_prep(dst2, n_pad)
    m2 = _mlp(u2, ef1, W1b[256:], b1b, W2b, b2b)
    part2 = _sc_scatter_max(m2, prep_b, n_pad).reshape(_NC, n_pad, 128)

    sel0 = part2[0, 8 : n - 1 : 9]
    sel1 = part2[1, 8 : n - 1 : 9]
    rows = sel0.shape[0]
    rows_pad = (rows + 7) // 8 * 8
    pad = rows_pad - rows
    sel0 = jnp.pad(sel0, ((0, pad), (0, 0)))
    sel1 = jnp.pad(sel1, ((0, pad), (0, 0)))
    o = _final(sel0, sel1, Wr, br)
    return o[:rows]
